# baseline (speedup 1.0000x reference)
import functools
import jax
import jax.numpy as jnp
from jax.experimental import pallas as pl
from jax.experimental.pallas import tpu as pltpu

N = 2048
C = 1024
H = 16
HD = 64
S = 64
NEG = -1e30


def _matmul_nt_kernel(x_ref, w_ref, b_ref, o_ref, *, out_dtype):
    xb = x_ref[...].astype(jnp.bfloat16)
    acc = jax.lax.dot_general(
        xb, w_ref[...], (((1,), (1,)), ((), ())),
        preferred_element_type=jnp.float32)
    o_ref[...] = (acc + b_ref[...]).astype(out_dtype)


def _matmul_nt(x, w_bf16, b, bm, bn, out_dtype):
    m, k = x.shape
    n = w_bf16.shape[0]
    kern = functools.partial(_matmul_nt_kernel, out_dtype=out_dtype)
    return pl.pallas_call(
        kern,
        grid=(m // bm, n // bn),
        in_specs=[
            pl.BlockSpec((bm, k), lambda i, j: (i, 0)),
            pl.BlockSpec((bn, k), lambda i, j: (j, 0)),
            pl.BlockSpec((1, bn), lambda i, j: (0, j)),
        ],
        out_specs=pl.BlockSpec((bm, bn), lambda i, j: (i, j)),
        out_shape=jax.ShapeDtypeStruct((m, n), out_dtype),
    )(x, w_bf16, b.reshape(1, n))


def _prep_kernel(oh_ref, hub_ref, adj_ref, fb_ref, qaug_ref, kaug_ref):
    rows = jax.lax.broadcasted_iota(jnp.int32, (S, S), 0)
    cols = jax.lax.broadcasted_iota(jnp.int32, (S, S), 1)
    allow_f = (adj_ref[...] > 0) | (rows == cols)
    table = jnp.where(allow_f, fb_ref[...], NEG)

    oh = oh_ref[...]
    r0 = jnp.dot(oh, table, preferred_element_type=jnp.float32)
    diag = jnp.sum(r0 * oh, axis=1, keepdims=True)
    hub = hub_ref[...] > 0.0
    own = oh > 0.0
    r = jnp.where(hub, jnp.where(own, diag, NEG), r0)
    cp = jnp.where(hub & ~own, NEG, 0.0)
    qaug_ref[:, :S] = r.astype(jnp.bfloat16)
    qaug_ref[:, S:] = oh.astype(jnp.bfloat16)
    kaug_ref[:, :S] = oh.astype(jnp.bfloat16)
    kaug_ref[:, S:] = cp.astype(jnp.bfloat16)


def _prep(frame_onehot, hub_col, adj, frame_bias):
    return pl.pallas_call(
        _prep_kernel,
        out_shape=(jax.ShapeDtypeStruct((N, 2 * S), jnp.bfloat16),
                   jax.ShapeDtypeStruct((N, 2 * S), jnp.bfloat16)),
    )(frame_onehot, hub_col, adj, frame_bias)


def _attn_kernel(q_ref, k_ref, v_ref, qaug_ref, kaug_ref, o_ref, mask_ref,
                 *, bq, bk, scale):
    nkb = N // bk

    qaug = qaug_ref[...]

    for kb in range(nkb):
        kaug = kaug_ref[pl.ds(kb * bk, bk), :]
        mask_ref[:, pl.ds(kb * bk, bk)] = jax.lax.dot_general(
            qaug, kaug, (((1,), (1,)), ((), ())),
            preferred_element_type=jnp.float32)

    for h in range(H):
        qh = q_ref[:, h * HD:(h + 1) * HD] * jnp.bfloat16(scale)
        m = jnp.zeros((bq, 1), jnp.float32)
        l = jnp.zeros((bq, 1), jnp.float32)
        o = jnp.zeros((bq, HD), jnp.float32)
        for kb in range(nkb):
            kh = k_ref[pl.ds(kb * bk, bk), h * HD:(h + 1) * HD]
            vh = v_ref[pl.ds(kb * bk, bk), h * HD:(h + 1) * HD]
            s = jax.lax.dot_general(
                qh, kh, (((1,), (1,)), ((), ())),
                preferred_element_type=jnp.float32)
            s = s + mask_ref[:, pl.ds(kb * bk, bk)]
            m_new = jnp.maximum(m, jnp.max(s, axis=1, keepdims=True))
            p = jnp.exp(s - m_new)
            alpha = jnp.exp(m - m_new)
            l = l * alpha + jnp.sum(p, axis=1, keepdims=True)
            o = o * alpha + jax.lax.dot_general(
                p.astype(jnp.bfloat16), vh, (((1,), (0,)), ((), ())),
                preferred_element_type=jnp.float32)
            m = m_new
        o_ref[:, h * HD:(h + 1) * HD] = (o / l).astype(jnp.bfloat16)


def _attention(qkv, qaug, kaug, bq=128, bk=2048):
    scale = HD ** (-0.5)
    kern = functools.partial(_attn_kernel, bq=bq, bk=bk, scale=scale)
    return pl.pallas_call(
        kern,
        grid=(N // bq,),
        scratch_shapes=[pltpu.VMEM((bq, N), jnp.float32)],
        in_specs=[
            pl.BlockSpec((bq, C), lambda i: (i, 0)),
            pl.BlockSpec((N, C), lambda i: (0, 1)),
            pl.BlockSpec((N, C), lambda i: (0, 2)),
            pl.BlockSpec((bq, 2 * S), lambda i: (i, 0)),
            pl.BlockSpec((N, 2 * S), lambda i: (0, 0)),
        ],
        out_specs=pl.BlockSpec((bq, C), lambda i: (i, 0)),
        out_shape=jax.ShapeDtypeStruct((N, C), jnp.bfloat16),
    )(qkv, qkv, qkv, qaug, kaug)


def kernel(x, frame_ids, is_hub, adj, frame_bias, Wqkv, bqkv, Wproj, bproj):
    Bx, Nx, Cx = x.shape
    x2 = x.reshape(Nx, Cx)

    fid = frame_ids.astype(jnp.int32)
    frame_onehot = (fid[:, None] ==
                    jnp.arange(S, dtype=jnp.int32)[None, :]).astype(jnp.float32)
    hub_col = (is_hub > 0).astype(jnp.float32).reshape(Nx, 1)

    qaug, kaug = _prep(frame_onehot, hub_col, adj, frame_bias)
    qkv = _matmul_nt(x2, Wqkv.astype(jnp.bfloat16), bqkv,
                     bm=2048, bn=512, out_dtype=jnp.bfloat16)
    attn_out = _attention(qkv, qaug, kaug)
    out = _matmul_nt(attn_out, Wproj.astype(jnp.bfloat16), bproj,
                     bm=2048, bn=512, out_dtype=jnp.float32)
    return out.reshape(Bx, Nx, Cx)


# speedup vs baseline: 1.2816x; 1.2816x over previous
import functools
import jax
import jax.numpy as jnp
from jax.experimental import pallas as pl
from jax.experimental.pallas import tpu as pltpu

N = 2048
C = 1024
H = 16
HD = 64
S = 64
NEG = -1e30
LOG2E = 1.4426950408889634


def _matmul_nt_kernel(x_ref, w_ref, b_ref, c_ref, o_ref, *, out_dtype):
    xb = x_ref[...].astype(jnp.bfloat16)
    acc = jax.lax.dot_general(
        xb, w_ref[...], (((1,), (1,)), ((), ())),
        preferred_element_type=jnp.float32)
    o_ref[...] = ((acc + b_ref[...]) * c_ref[...]).astype(out_dtype)


def _matmul_nt(x, w_bf16, b, colscale, bm, bn, out_dtype):
    m, k = x.shape
    n = w_bf16.shape[0]
    kern = functools.partial(_matmul_nt_kernel, out_dtype=out_dtype)
    return pl.pallas_call(
        kern,
        grid=(m // bm, n // bn),
        in_specs=[
            pl.BlockSpec((bm, k), lambda i, j: (i, 0)),
            pl.BlockSpec((bn, k), lambda i, j: (j, 0)),
            pl.BlockSpec((1, bn), lambda i, j: (0, j)),
            pl.BlockSpec((1, bn), lambda i, j: (0, j)),
        ],
        out_specs=pl.BlockSpec((bm, bn), lambda i, j: (i, j)),
        out_shape=jax.ShapeDtypeStruct((m, n), out_dtype),
    )(x, w_bf16, b.reshape(1, n), colscale.reshape(1, n))


def _prep_kernel(oh_ref, hub_ref, adj_ref, fb_ref, qaug_ref, kaug_ref):
    rows = jax.lax.broadcasted_iota(jnp.int32, (S, S), 0)
    cols = jax.lax.broadcasted_iota(jnp.int32, (S, S), 1)
    allow_f = (adj_ref[...] > 0) | (rows == cols)
    table = jnp.where(allow_f, fb_ref[...], NEG)

    oh = oh_ref[...]
    r0 = jnp.dot(oh, table, preferred_element_type=jnp.float32)
    diag = jnp.sum(r0 * oh, axis=1, keepdims=True)
    hub = hub_ref[...] > 0.0
    own = oh > 0.0
    r = jnp.where(hub, jnp.where(own, diag, NEG), r0) * LOG2E
    cp = jnp.where(hub & ~own, NEG * LOG2E, 0.0)
    qaug_ref[:, :S] = r.astype(jnp.bfloat16)
    qaug_ref[:, S:] = oh.astype(jnp.bfloat16)
    kaug_ref[:, :S] = oh.astype(jnp.bfloat16)
    kaug_ref[:, S:] = cp.astype(jnp.bfloat16)


def _prep(frame_onehot, hub_col, adj, frame_bias):
    return pl.pallas_call(
        _prep_kernel,
        out_shape=(jax.ShapeDtypeStruct((N, 2 * S), jnp.bfloat16),
                   jax.ShapeDtypeStruct((N, 2 * S), jnp.bfloat16)),
    )(frame_onehot, hub_col, adj, frame_bias)


def _attn_kernel(q_ref, k_ref, v_ref, qaug_ref, kaug_ref, o_ref, mask_ref,
                 *, bq, bk, scale):
    nkb = N // bk

    qaug = qaug_ref[...]

    for kb in range(nkb):
        kaug = kaug_ref[pl.ds(kb * bk, bk), :]
        mask_ref[:, pl.ds(kb * bk, bk)] = jax.lax.dot_general(
            qaug, kaug, (((1,), (1,)), ((), ())),
            preferred_element_type=jnp.float32)

    for h in range(H):
        qh = q_ref[:, h * HD:(h + 1) * HD]
        m = jnp.zeros((bq, 1), jnp.float32)
        l = jnp.zeros((bq, 1), jnp.float32)
        o = jnp.zeros((bq, HD), jnp.float32)
        for kb in range(nkb):
            kh = k_ref[pl.ds(kb * bk, bk), h * HD:(h + 1) * HD]
            vh = v_ref[pl.ds(kb * bk, bk), h * HD:(h + 1) * HD]
            s = jax.lax.dot_general(
                qh, kh, (((1,), (1,)), ((), ())),
                preferred_element_type=jnp.float32)
            s = s + mask_ref[:, pl.ds(kb * bk, bk)]
            m_new = jnp.maximum(m, jnp.max(s, axis=1, keepdims=True))
            p = jnp.exp2(s - m_new)
            alpha = jnp.exp2(m - m_new)
            l = l * alpha + jnp.sum(p, axis=1, keepdims=True)
            o = o * alpha + jax.lax.dot_general(
                p.astype(jnp.bfloat16), vh, (((1,), (0,)), ((), ())),
                preferred_element_type=jnp.float32)
            m = m_new
        o_ref[:, h * HD:(h + 1) * HD] = (o / l).astype(jnp.bfloat16)


def _attention(qkv, qaug, kaug, bq=256, bk=2048):
    scale = HD ** (-0.5)
    kern = functools.partial(_attn_kernel, bq=bq, bk=bk, scale=scale)
    return pl.pallas_call(
        kern,
        grid=(N // bq,),
        scratch_shapes=[pltpu.VMEM((bq, N), jnp.float32)],
        in_specs=[
            pl.BlockSpec((bq, C), lambda i: (i, 0)),
            pl.BlockSpec((N, C), lambda i: (0, 1)),
            pl.BlockSpec((N, C), lambda i: (0, 2)),
            pl.BlockSpec((bq, 2 * S), lambda i: (i, 0)),
            pl.BlockSpec((N, 2 * S), lambda i: (0, 0)),
        ],
        out_specs=pl.BlockSpec((bq, C), lambda i: (i, 0)),
        out_shape=jax.ShapeDtypeStruct((N, C), jnp.bfloat16),
    )(qkv, qkv, qkv, qaug, kaug)


def kernel(x, frame_ids, is_hub, adj, frame_bias, Wqkv, bqkv, Wproj, bproj):
    Bx, Nx, Cx = x.shape
    x2 = x.reshape(Nx, Cx)

    fid = frame_ids.astype(jnp.int32)
    frame_onehot = (fid[:, None] ==
                    jnp.arange(S, dtype=jnp.int32)[None, :]).astype(jnp.float32)
    hub_col = (is_hub > 0).astype(jnp.float32).reshape(Nx, 1)

    qs = (HD ** (-0.5)) * LOG2E
    colscale = jnp.concatenate([
        jnp.full((Cx,), qs, jnp.float32),
        jnp.ones((2 * Cx,), jnp.float32)])

    qaug, kaug = _prep(frame_onehot, hub_col, adj, frame_bias)
    qkv = _matmul_nt(x2, Wqkv.astype(jnp.bfloat16), bqkv, colscale,
                     bm=2048, bn=512, out_dtype=jnp.bfloat16)
    attn_out = _attention(qkv, qaug, kaug)
    out = _matmul_nt(attn_out, Wproj.astype(jnp.bfloat16), bproj,
                     jnp.ones((Cx,), jnp.float32),
                     bm=2048, bn=512, out_dtype=jnp.float32)
    return out.reshape(Bx, Nx, Cx)
